# Initial kernel scaffold; baseline (speedup 1.0000x reference)
import jax
import jax.numpy as jnp
from jax.experimental import pallas as pl


def kernel(x, W, a_src, a_dst, mid_table, big_table, edge_index, sub_ids, mid_ids, big_ids):
    raise NotImplementedError("write your pallas kernel here")


# speedup vs baseline: 9.5973x; 9.5973x over previous
import functools

import jax
import jax.numpy as jnp
from jax import lax
from jax.experimental import pallas as pl
from jax.experimental.pallas import tpu as pltpu
from jax.experimental.pallas import tpu_sc as plsc

N = 10000
E = 320000
D = 128
DH = D // 2
NSUB = 256
NMID = 64
NBIG = 16

NC = 2
NS = 16
NW = NC * NS
EPT = E // NW
EPS = E // NS
CH = 80
RCH = 80
NRC = N // RCH
KM = (NRC + NS - 1) // NS

_MESH = plsc.VectorSubcoreMesh(core_axis_name="c", subcore_axis_name="s",
                               num_cores=NC, num_subcores=NS)
_SCP = pltpu.CompilerParams(needs_layout_passes=False,
                            use_tc_tiling_on_sc=False)

_f32 = jnp.float32
_i32 = jnp.int32


def _tch_body(x_ref, w_ref, asrc_ref, adst_ref, h_ref, hs_ref, hd_ref):
    h = jnp.dot(x_ref[...], w_ref[...], preferred_element_type=_f32)
    h_ref[...] = h
    hs_ref[...] = jnp.dot(h, asrc_ref[...], preferred_element_type=_f32)
    hd_ref[...] = jnp.dot(h, adst_ref[...], preferred_element_type=_f32)


def _tch(x, W, asrc2, adst2):
    blk = 1000
    h, hs, hd = pl.pallas_call(
        _tch_body,
        grid=(N // blk,),
        in_specs=[pl.BlockSpec((blk, D), lambda i: (i, 0)),
                  pl.BlockSpec((D, D), lambda i: (0, 0)),
                  pl.BlockSpec((D, 1), lambda i: (0, 0)),
                  pl.BlockSpec((D, 1), lambda i: (0, 0))],
        out_specs=[pl.BlockSpec((blk, D), lambda i: (i, 0)),
                   pl.BlockSpec((blk, 1), lambda i: (i, 0)),
                   pl.BlockSpec((blk, 1), lambda i: (i, 0))],
        out_shape=[jax.ShapeDtypeStruct((N, D), _f32),
                   jax.ShapeDtypeStruct((N, 1), _f32),
                   jax.ShapeDtypeStruct((N, 1), _f32)],
    )(x, W, asrc2, adst2)
    return h, hs.reshape(N), hd.reshape(N)


def _tcinv_body(denp_ref, inv_ref):
    inv_ref[...] = 1.0 / (jnp.sum(denp_ref[...], axis=0, keepdims=True) + 1e-16)


def _tcinv(denp):
    inv = pl.pallas_call(
        _tcinv_body,
        out_shape=jax.ShapeDtypeStruct((1, N), _f32),
    )(denp)
    return inv.reshape(N)


def _gathertc_body(n_parent, mm_ref, pid_ref, o_ref):
    ids = pid_ref[0, 0, :]
    onehot = (ids[:, None] ==
              lax.broadcasted_iota(_i32, (ids.shape[0], n_parent), 1)).astype(_f32)
    o_ref[...] = jnp.dot(onehot, mm_ref[...], preferred_element_type=_f32)


def _gathertc(mm, pid, n_parent):
    blk = 1000
    pid3 = pid.reshape(N // blk, 1, blk)
    out = pl.pallas_call(
        functools.partial(_gathertc_body, n_parent),
        grid=(N // blk,),
        in_specs=[pl.BlockSpec((n_parent, D), lambda i: (0, 0)),
                  pl.BlockSpec((1, 1, blk), lambda i: (i, 0, 0))],
        out_specs=pl.BlockSpec((blk, D), lambda i: (i, 0)),
        out_shape=jax.ShapeDtypeStruct((N, D), _f32),
    )(mm, pid3)
    return out


def _tcfin_body(s0, m0, b0, x_ref, o_ref):
    o_ref[0] = s0[...]
    o_ref[1] = m0[...]
    o_ref[2] = b0[...] + x_ref[...]


def _tcfin(sub, mid2, big2, x):
    blk = 1000
    spec = pl.BlockSpec((blk, D), lambda i: (i, 0))
    return pl.pallas_call(
        _tcfin_body,
        grid=(N // blk,),
        in_specs=[spec] * 4,
        out_specs=pl.BlockSpec((3, blk, D), lambda i: (0, i, 0)),
        out_shape=jax.ShapeDtypeStruct((3, N, D), _f32),
    )(sub, mid2, big2, x)


def _sca_body(src_hbm, dst_hbm, hs_hbm, hd_hbm, ex_hbm, denp_hbm,
              hs_v, hd_v, src_v, dst_v, ex_v, den_v):
    c = lax.axis_index("c")
    s = lax.axis_index("s")
    wid = s * NC + c
    base = wid * EPT

    pltpu.sync_copy(hs_hbm, hs_v)
    pltpu.sync_copy(hd_hbm, hd_v)
    pltpu.sync_copy(src_hbm.at[pl.ds(base, EPT)], src_v)
    pltpu.sync_copy(dst_hbm.at[pl.ds(base, EPT)], dst_v)

    z16 = jnp.zeros((16,), _f32)

    def zbody(i, carry):
        den_v[pl.ds(i * 16, 16)] = z16
        return carry
    lax.fori_loop(0, N // 16, zbody, 0)

    def ebody(i, carry):
        s16 = src_v[pl.ds(i * 16, 16)]
        d16 = dst_v[pl.ds(i * 16, 16)]
        a = plsc.load_gather(hs_v, [s16])
        b = plsc.load_gather(hd_v, [d16])
        sc = a + b
        sc = jnp.where(sc >= 0.0, sc, sc * 0.2)
        e16 = jnp.exp(sc)
        ex_v[pl.ds(i * 16, 16)] = e16
        plsc.addupdate_scatter(den_v, [d16], e16)
        return carry
    lax.fori_loop(0, EPT // 16, ebody, 0)

    pltpu.sync_copy(ex_v, ex_hbm.at[pl.ds(base, EPT)])
    pltpu.sync_copy(den_v, denp_hbm.at[wid, 0])


_sca = pl.kernel(
    _sca_body,
    out_type=(jax.ShapeDtypeStruct((E,), _f32),
              jax.ShapeDtypeStruct((NW, 1, N), _f32)),
    mesh=_MESH,
    compiler_params=_SCP,
    scratch_types=[
        pltpu.VMEM((N,), _f32),
        pltpu.VMEM((N,), _f32),
        pltpu.VMEM((EPT,), _i32),
        pltpu.VMEM((EPT,), _i32),
        pltpu.VMEM((EPT,), _f32),
        pltpu.VMEM((N,), _f32),
    ],
)


def _scb_body(h2_hbm, src_hbm, dst_hbm, ex_hbm, inv_hbm, out_hbm,
              inv_v, src_v, dst_v, al_v, dstc_v, rows_v, acc, gsem):
    c = lax.axis_index("c")
    s = lax.axis_index("s")
    base = s * EPS

    pltpu.sync_copy(inv_hbm, inv_v)
    pltpu.sync_copy(src_hbm.at[pl.ds(base, EPS)], src_v)
    pltpu.sync_copy(dst_hbm.at[pl.ds(base, EPS)], dst_v)
    pltpu.sync_copy(ex_hbm.at[pl.ds(base, EPS)], al_v)

    def abody(i, carry):
        d16 = dst_v[pl.ds(i * 16, 16)]
        iv = plsc.load_gather(inv_v, [d16])
        al_v[pl.ds(i * 16, 16)] = al_v[pl.ds(i * 16, 16)] * iv
        src_v[pl.ds(i * 16, 16)] = src_v[pl.ds(i * 16, 16)] * 2 + c
        return carry
    lax.fori_loop(0, EPS // 16, abody, 0)

    z16 = jnp.zeros((16,), _f32)

    def zrow(i, carry):
        for j in range(DH // 16):
            rows_v[i, pl.ds(j * 16, 16)] = z16
        return carry
    lax.fori_loop(0, CH, zrow, 0)
    for k in range(KM):
        ck = s + NS * k

        @pl.when(ck < NRC)
        def _():
            pltpu.sync_copy(rows_v, acc.at[pl.ds(ck * RCH, RCH)])
    plsc.subcore_barrier()

    def chunk(g, carry):
        pltpu.async_copy(h2_hbm.at[src_v.at[pl.ds(g * CH, CH)]], rows_v,
                         gsem).wait()

        def cb(k2, cc):
            dstc_v[pl.ds(k2 * 16, 16)] = dst_v[pl.ds(g * CH + k2 * 16, 16)]
            return cc
        lax.fori_loop(0, CH // 16, cb, 0)

        def srow(i, cc):
            a16 = plsc.load_gather(al_v, [jnp.full((16,), g * CH + i, _i32)])
            for j in range(DH // 16):
                rows_v[i, pl.ds(j * 16, 16)] = rows_v[i, pl.ds(j * 16, 16)] * a16
            return cc
        lax.fori_loop(0, CH, srow, 0)

        pltpu.sync_copy(rows_v, acc.at[dstc_v], add=True)
        return carry
    lax.fori_loop(0, EPS // CH, chunk, 0)
    plsc.subcore_barrier()

    for k in range(KM):
        ck = s + NS * k

        @pl.when(ck < NRC)
        def _():
            pltpu.sync_copy(acc.at[pl.ds(ck * RCH, RCH)],
                            out_hbm.at[pl.ds(ck * RCH, RCH), c, 0])


_scb = pl.kernel(
    _scb_body,
    out_type=jax.ShapeDtypeStruct((N, NC, 1, DH), _f32),
    mesh=_MESH,
    compiler_params=_SCP,
    scratch_types=[
        pltpu.VMEM((N,), _f32),
        pltpu.VMEM((EPS,), _i32),
        pltpu.VMEM((EPS,), _i32),
        pltpu.VMEM((EPS,), _f32),
        pltpu.VMEM((CH,), _i32),
        pltpu.VMEM((CH, DH), _f32),
        pltpu.VMEM_SHARED((N, DH), _f32),
        pltpu.SemaphoreType.DMA,
    ],
)


def _agg_body(n_parent, n_child, accr,
              p_hbm, tbl_hbm, cid_hbm, pid_hbm, mm_hbm,
              zc_v, cred_v, cidc_v, pidc_v, pkl_v, pkf_v, ones_v,
              rb_v, mmrow_v, tbl_v, psum_v, acc, cnt_sh):
    nsp = n_parent * n_child
    quarter = nsp // 4
    ppq = n_parent // 4
    ppt = max(1, ppq // NS)
    ppn = ppt * n_child
    nzc = (accr + RCH * NS - 1) // (RCH * NS)
    csl = nsp // NS

    c = lax.axis_index("c")
    s = lax.axis_index("s")

    pltpu.sync_copy(tbl_hbm, tbl_v)

    z16 = jnp.zeros((16,), _f32)
    one16 = jnp.ones((16,), _f32)

    def zrow(i, carry):
        for j in range(D // 16):
            rb_v[i, pl.ds(j * 16, 16)] = z16
        return carry
    lax.fori_loop(0, RCH, zrow, 0)

    def zo(i, carry):
        ones_v[pl.ds(i * 16, 16)] = one16
        return carry
    lax.fori_loop(0, RCH // 16, zo, 0)

    for q in range(2):
        for k in range(nzc):
            ck = s + NS * k

            @pl.when(ck * RCH < accr)
            def _():
                pltpu.sync_copy(rb_v, acc.at[pl.ds(ck * RCH, RCH)])
        if q == 0:
            def zc(i, carry):
                zc_v[pl.ds(i * 16, 16)] = z16
                return carry
            lax.fori_loop(0, csl // 16, zc, 0)
            pltpu.sync_copy(zc_v, cnt_sh.at[pl.ds(s * csl, csl)])
        plsc.subcore_barrier()

        lo = (c * 2 + q) * quarter
        for k in range(KM):
            ck = s + NS * k

            @pl.when(ck < NRC)
            def _():
                r0 = ck * RCH
                pltpu.sync_copy(cid_hbm.at[pl.ds(r0, RCH)], cidc_v)
                pltpu.sync_copy(pid_hbm.at[pl.ds(r0, RCH)], pidc_v)
                pltpu.sync_copy(p_hbm.at[pl.ds(r0, RCH)], rb_v)

                def pkb(t, cc):
                    ci = cidc_v[pl.ds(t * 16, 16)]
                    pi = pidc_v[pl.ds(t * 16, 16)]
                    pk = pi * n_child + ci
                    pkf_v[pl.ds(t * 16, 16)] = pk
                    inq = (pk >= lo) & (pk < lo + quarter)
                    pkl_v[pl.ds(t * 16, 16)] = jnp.where(inq, pk - lo, quarter)
                    return cc
                lax.fori_loop(0, RCH // 16, pkb, 0)

                if q == 0:
                    pltpu.sync_copy(ones_v, cnt_sh.at[pkf_v], add=True)
                pltpu.sync_copy(rb_v, acc.at[pkl_v], add=True)
        plsc.subcore_barrier()

        @pl.when(s * ppt < ppq)
        def _():
            gp0 = lo + s * ppn
            lp0 = s * ppn

            pltpu.sync_copy(cnt_sh.at[pl.ds(gp0, ppn)], cred_v)
            pltpu.sync_copy(acc.at[pl.ds(lp0, ppn)], psum_v)

            for pp in range(ppt):
                gpar = (c * 2 + q) * ppq + s * ppt + pp

                def prow(r, carry):
                    vecs = carry[:-1]
                    mcnt = carry[-1]
                    li = pp * n_child + r
                    cnt16 = plsc.load_gather(cred_v,
                                             [jnp.full((16,), li, _i32)])
                    present = cnt16 > 0.0
                    inv16 = jnp.where(present,
                                      1.0 / jnp.maximum(cnt16, 1.0), 0.0)
                    new = []
                    for j in range(D // 16):
                        pj = psum_v[li, pl.ds(j * 16, 16)]
                        tj = tbl_v[gpar, pl.ds(j * 16, 16)]
                        new.append(vecs[j] + jnp.where(present,
                                                       pj * inv16 + tj, 0.0))
                    mcnt = mcnt + jnp.where(present, 1.0, 0.0)
                    return tuple(new) + (mcnt,)

                init = tuple(jnp.zeros((16,), _f32)
                             for _ in range(D // 16 + 1))
                res = lax.fori_loop(0, n_child, prow, init)
                minv = 1.0 / jnp.maximum(res[-1], 1.0)
                for j in range(D // 16):
                    mmrow_v[0, pl.ds(j * 16, 16)] = 1.0 / (
                        1.0 + jnp.exp(-res[j] * minv))
                pltpu.sync_copy(mmrow_v, mm_hbm.at[gpar])
        plsc.subcore_barrier()


def _make_agg(n_parent, n_child):
    nsp = n_parent * n_child
    quarter = nsp // 4
    accr = -(-(quarter + 8) // RCH) * RCH
    ppn = max(1, (n_parent // 4) // NS) * n_child
    return pl.kernel(
        functools.partial(_agg_body, n_parent, n_child, accr),
        out_type=jax.ShapeDtypeStruct((n_parent, 1, D), _f32),
        mesh=_MESH,
        compiler_params=_SCP,
        scratch_types=[
            pltpu.VMEM((nsp // NS,), _f32),
            pltpu.VMEM((ppn,), _f32),
            pltpu.VMEM((RCH,), _i32),
            pltpu.VMEM((RCH,), _i32),
            pltpu.VMEM((RCH,), _i32),
            pltpu.VMEM((RCH,), _i32),
            pltpu.VMEM((RCH,), _f32),
            pltpu.VMEM((RCH, D), _f32),
            pltpu.VMEM((1, D), _f32),
            pltpu.VMEM((n_parent, D), _f32),
            pltpu.VMEM((ppn, D), _f32),
            pltpu.VMEM_SHARED((accr, D), _f32),
            pltpu.VMEM_SHARED((nsp,), _f32),
        ],
    )


_agg_mid = _make_agg(NMID, NSUB)
_agg_big = _make_agg(NBIG, NMID)


def _gat_sc(x, W, asrc2, adst2, src, dst):
    h, hs, hd = _tch(x, W, asrc2, adst2)
    ex, denp = _sca(src, dst, hs, hd)
    inv = _tcinv(denp.reshape(NW, N))
    out4 = _scb(h.reshape(2 * N, DH), src, dst, ex, inv)
    return out4.reshape(N, D)


def kernel(x, W, a_src, a_dst, mid_table, big_table, edge_index, sub_ids,
           mid_ids, big_ids):
    src = edge_index[0]
    dst = edge_index[1]
    asrc2 = a_src.reshape(D, 1)
    adst2 = a_dst.reshape(D, 1)

    sub_emb = _gat_sc(x, W, asrc2, adst2, src, dst)
    mm_mid = _agg_mid(sub_emb, mid_table, sub_ids, mid_ids)
    mid_emb = _gathertc(mm_mid.reshape(NMID, D), mid_ids, NMID)

    mid2 = _gat_sc(mid_emb, W, asrc2, adst2, src, dst)
    mm_big = _agg_big(mid2, big_table, mid_ids, big_ids)
    big_emb = _gathertc(mm_big.reshape(NBIG, D), big_ids, NBIG)

    big2 = _gat_sc(big_emb, W, asrc2, adst2, src, dst)
    return _tcfin(sub_emb, mid2, big2, x)


# speedup vs baseline: 14.2010x; 1.4797x over previous
import functools

import jax
import jax.numpy as jnp
from jax import lax
from jax.experimental import pallas as pl
from jax.experimental.pallas import tpu as pltpu
from jax.experimental.pallas import tpu_sc as plsc

N = 10000
E = 320000
D = 128
DH = D // 2
NSUB = 256
NMID = 64
NBIG = 16

NC = 2
NS = 16
NW = NC * NS
EPT = E // NW
EPS = E // NS
CH = 80
RCH = 80
NRC = N // RCH
KM = (NRC + NS - 1) // NS

_MESH = plsc.VectorSubcoreMesh(core_axis_name="c", subcore_axis_name="s",
                               num_cores=NC, num_subcores=NS)
_SCP = pltpu.CompilerParams(needs_layout_passes=False,
                            use_tc_tiling_on_sc=False)

_f32 = jnp.float32
_i32 = jnp.int32


def _tch_body(x_ref, w_ref, asrc_ref, adst_ref, h_ref, hs_ref, hd_ref):
    h = jnp.dot(x_ref[...], w_ref[...], preferred_element_type=_f32)
    h_ref[...] = h
    hs_ref[...] = jnp.dot(h, asrc_ref[...], preferred_element_type=_f32)
    hd_ref[...] = jnp.dot(h, adst_ref[...], preferred_element_type=_f32)


def _tch(x, W, asrc2, adst2):
    blk = 1000
    h, hs, hd = pl.pallas_call(
        _tch_body,
        grid=(N // blk,),
        in_specs=[pl.BlockSpec((blk, D), lambda i: (i, 0)),
                  pl.BlockSpec((D, D), lambda i: (0, 0)),
                  pl.BlockSpec((D, 1), lambda i: (0, 0)),
                  pl.BlockSpec((D, 1), lambda i: (0, 0))],
        out_specs=[pl.BlockSpec((blk, D), lambda i: (i, 0)),
                   pl.BlockSpec((blk, 1), lambda i: (i, 0)),
                   pl.BlockSpec((blk, 1), lambda i: (i, 0))],
        out_shape=[jax.ShapeDtypeStruct((N, D), _f32),
                   jax.ShapeDtypeStruct((N, 1), _f32),
                   jax.ShapeDtypeStruct((N, 1), _f32)],
    )(x, W, asrc2, adst2)
    return h, hs.reshape(N), hd.reshape(N)


def _tcinv_body(denp_ref, inv_ref):
    inv_ref[...] = 1.0 / (jnp.sum(denp_ref[...], axis=0, keepdims=True) + 1e-16)


def _tcinv(denp):
    inv = pl.pallas_call(
        _tcinv_body,
        out_shape=jax.ShapeDtypeStruct((1, N), _f32),
    )(denp)
    return inv.reshape(N)


def _gathertc_body(n_parent, mm_ref, pid_ref, o_ref):
    ids = pid_ref[0, 0, :]
    onehot = (ids[:, None] ==
              lax.broadcasted_iota(_i32, (ids.shape[0], n_parent), 1)).astype(_f32)
    o_ref[...] = jnp.dot(onehot, mm_ref[...], preferred_element_type=_f32)


def _gathertc(mm, pid, n_parent):
    blk = 1000
    pid3 = pid.reshape(N // blk, 1, blk)
    out = pl.pallas_call(
        functools.partial(_gathertc_body, n_parent),
        grid=(N // blk,),
        in_specs=[pl.BlockSpec((n_parent, D), lambda i: (0, 0)),
                  pl.BlockSpec((1, 1, blk), lambda i: (i, 0, 0))],
        out_specs=pl.BlockSpec((blk, D), lambda i: (i, 0)),
        out_shape=jax.ShapeDtypeStruct((N, D), _f32),
    )(mm, pid3)
    return out


def _tcfin_body(s0, m0, b0, x_ref, o_ref):
    o_ref[0] = s0[...]
    o_ref[1] = m0[...]
    o_ref[2] = b0[...] + x_ref[...]


def _tcfin(sub, mid2, big2, x):
    blk = 1000
    spec = pl.BlockSpec((blk, D), lambda i: (i, 0))
    return pl.pallas_call(
        _tcfin_body,
        grid=(N // blk,),
        in_specs=[spec] * 4,
        out_specs=pl.BlockSpec((3, blk, D), lambda i: (0, i, 0)),
        out_shape=jax.ShapeDtypeStruct((3, N, D), _f32),
    )(sub, mid2, big2, x)


def _sca_body(src_hbm, dst_hbm, hs_hbm, hd_hbm, ex_hbm, denp_hbm,
              hs_v, hd_v, src_v, dst_v, ex_v, den_v):
    c = lax.axis_index("c")
    s = lax.axis_index("s")
    wid = s * NC + c
    base = wid * EPT

    pltpu.sync_copy(hs_hbm, hs_v)
    pltpu.sync_copy(hd_hbm, hd_v)
    pltpu.sync_copy(src_hbm.at[pl.ds(base, EPT)], src_v)
    pltpu.sync_copy(dst_hbm.at[pl.ds(base, EPT)], dst_v)

    z16 = jnp.zeros((16,), _f32)

    def zbody(i, carry):
        den_v[pl.ds(i * 16, 16)] = z16
        return carry
    lax.fori_loop(0, N // 16, zbody, 0)

    def ebody(i, carry):
        s16 = src_v[pl.ds(i * 16, 16)]
        d16 = dst_v[pl.ds(i * 16, 16)]
        a = plsc.load_gather(hs_v, [s16])
        b = plsc.load_gather(hd_v, [d16])
        sc = a + b
        sc = jnp.where(sc >= 0.0, sc, sc * 0.2)
        e16 = jnp.exp(sc)
        ex_v[pl.ds(i * 16, 16)] = e16
        plsc.addupdate_scatter(den_v, [d16], e16)
        return carry
    lax.fori_loop(0, EPT // 16, ebody, 0)

    pltpu.sync_copy(ex_v, ex_hbm.at[pl.ds(base, EPT)])
    pltpu.sync_copy(den_v, denp_hbm.at[wid, 0])


_sca = pl.kernel(
    _sca_body,
    out_type=(jax.ShapeDtypeStruct((E,), _f32),
              jax.ShapeDtypeStruct((NW, 1, N), _f32)),
    mesh=_MESH,
    compiler_params=_SCP,
    scratch_types=[
        pltpu.VMEM((N,), _f32),
        pltpu.VMEM((N,), _f32),
        pltpu.VMEM((EPT,), _i32),
        pltpu.VMEM((EPT,), _i32),
        pltpu.VMEM((EPT,), _f32),
        pltpu.VMEM((N,), _f32),
    ],
)


def _scb_body(h2_hbm, src_hbm, dst_hbm, ex_hbm, inv_hbm, out_hbm,
              inv_v, src_v, dst_v, al_v, dstc_v, rows_v, acc,
              gsem0, gsem1, ssem0, ssem1):
    gsems = (gsem0, gsem1)
    ssems = (ssem0, ssem1)
    c = lax.axis_index("c")
    s = lax.axis_index("s")
    base = s * EPS

    pltpu.sync_copy(inv_hbm, inv_v)
    pltpu.sync_copy(src_hbm.at[pl.ds(base, EPS)], src_v)
    pltpu.sync_copy(dst_hbm.at[pl.ds(base, EPS)], dst_v)
    pltpu.sync_copy(ex_hbm.at[pl.ds(base, EPS)], al_v)

    def abody(i, carry):
        d16 = dst_v[pl.ds(i * 16, 16)]
        iv = plsc.load_gather(inv_v, [d16])
        al_v[pl.ds(i * 16, 16)] = al_v[pl.ds(i * 16, 16)] * iv
        src_v[pl.ds(i * 16, 16)] = src_v[pl.ds(i * 16, 16)] * 2 + c
        return carry
    lax.fori_loop(0, EPS // 16, abody, 0)

    z16 = jnp.zeros((16,), _f32)

    def zrow(i, carry):
        for j in range(DH // 16):
            rows_v[0, i, pl.ds(j * 16, 16)] = z16
        return carry
    lax.fori_loop(0, CH, zrow, 0)
    for k in range(KM):
        ck = s + NS * k

        @pl.when(ck < NRC)
        def _():
            pltpu.sync_copy(rows_v.at[0], acc.at[pl.ds(ck * RCH, RCH)])
    plsc.subcore_barrier()

    nchk = EPS // CH

    def _gather(g, b):
        pltpu.async_copy(h2_hbm.at[src_v.at[pl.ds(g * CH, CH)]],
                         rows_v.at[b], gsems[b])

    def _gather_wait(g, b):
        pltpu.make_async_copy(h2_hbm.at[src_v.at[pl.ds(g * CH, CH)]],
                              rows_v.at[b], gsems[b]).wait()

    def _scatter(b):
        pltpu.async_copy(rows_v.at[b], acc.at[dstc_v.at[b]],
                         ssems[b], add=True)

    def _scatter_wait(b):
        pltpu.make_async_copy(rows_v.at[b], acc.at[dstc_v.at[b]],
                              ssems[b]).wait()

    _gather(0, 0)

    def pair(it, carry):
        for b in range(2):
            g = it * 2 + b
            bp = 1 - b

            @pl.when(g + 1 < nchk)
            def _():
                @pl.when(g >= 1)
                def _():
                    _scatter_wait(bp)
                _gather(g + 1, bp)

            _gather_wait(g, b)

            def cb(k2, cc):
                dstc_v[b, pl.ds(k2 * 16, 16)] = dst_v[pl.ds(g * CH + k2 * 16,
                                                            16)]
                return cc
            lax.fori_loop(0, CH // 16, cb, 0)

            def srow(i, cc):
                a16 = plsc.load_gather(al_v,
                                       [jnp.full((16,), g * CH + i, _i32)])
                for j in range(DH // 16):
                    rows_v[b, i, pl.ds(j * 16, 16)] = (
                        rows_v[b, i, pl.ds(j * 16, 16)] * a16)
                return cc
            lax.fori_loop(0, CH, srow, 0)

            _scatter(b)
        return carry
    lax.fori_loop(0, nchk // 2, pair, 0)
    _scatter_wait(0)
    _scatter_wait(1)
    plsc.subcore_barrier()

    for k in range(KM):
        ck = s + NS * k

        @pl.when(ck < NRC)
        def _():
            pltpu.sync_copy(acc.at[pl.ds(ck * RCH, RCH)],
                            out_hbm.at[pl.ds(ck * RCH, RCH), c, 0])


_scb = pl.kernel(
    _scb_body,
    out_type=jax.ShapeDtypeStruct((N, NC, 1, DH), _f32),
    mesh=_MESH,
    compiler_params=_SCP,
    scratch_types=[
        pltpu.VMEM((N,), _f32),
        pltpu.VMEM((EPS,), _i32),
        pltpu.VMEM((EPS,), _i32),
        pltpu.VMEM((EPS,), _f32),
        pltpu.VMEM((2, CH), _i32),
        pltpu.VMEM((2, CH, DH), _f32),
        pltpu.VMEM_SHARED((N, DH), _f32),
        pltpu.SemaphoreType.DMA,
        pltpu.SemaphoreType.DMA,
        pltpu.SemaphoreType.DMA,
        pltpu.SemaphoreType.DMA,
    ],
)


def _agg_body(n_parent, n_child, accr,
              p_hbm, tbl_hbm, cid_hbm, pid_hbm, mm_hbm,
              zc_v, cred_v, cidc_v, pidc_v, pkl_v, pkf_v, ones_v,
              rb_v, mmrow_v, tbl_v, psum_v, acc, cnt_sh):
    nsp = n_parent * n_child
    quarter = nsp // 4
    ppq = n_parent // 4
    ppt = max(1, ppq // NS)
    ppn = ppt * n_child
    nzc = (accr + RCH * NS - 1) // (RCH * NS)
    csl = nsp // NS

    c = lax.axis_index("c")
    s = lax.axis_index("s")

    pltpu.sync_copy(tbl_hbm, tbl_v)

    z16 = jnp.zeros((16,), _f32)
    one16 = jnp.ones((16,), _f32)

    def zrow(i, carry):
        for j in range(D // 16):
            rb_v[i, pl.ds(j * 16, 16)] = z16
        return carry
    lax.fori_loop(0, RCH, zrow, 0)

    def zo(i, carry):
        ones_v[pl.ds(i * 16, 16)] = one16
        return carry
    lax.fori_loop(0, RCH // 16, zo, 0)

    for q in range(2):
        for k in range(nzc):
            ck = s + NS * k

            @pl.when(ck * RCH < accr)
            def _():
                pltpu.sync_copy(rb_v, acc.at[pl.ds(ck * RCH, RCH)])
        if q == 0:
            def zc(i, carry):
                zc_v[pl.ds(i * 16, 16)] = z16
                return carry
            lax.fori_loop(0, csl // 16, zc, 0)
            pltpu.sync_copy(zc_v, cnt_sh.at[pl.ds(s * csl, csl)])
        plsc.subcore_barrier()

        lo = (c * 2 + q) * quarter
        for k in range(KM):
            ck = s + NS * k

            @pl.when(ck < NRC)
            def _():
                r0 = ck * RCH
                pltpu.sync_copy(cid_hbm.at[pl.ds(r0, RCH)], cidc_v)
                pltpu.sync_copy(pid_hbm.at[pl.ds(r0, RCH)], pidc_v)
                pltpu.sync_copy(p_hbm.at[pl.ds(r0, RCH)], rb_v)

                def pkb(t, cc):
                    ci = cidc_v[pl.ds(t * 16, 16)]
                    pi = pidc_v[pl.ds(t * 16, 16)]
                    pk = pi * n_child + ci
                    pkf_v[pl.ds(t * 16, 16)] = pk
                    inq = (pk >= lo) & (pk < lo + quarter)
                    pkl_v[pl.ds(t * 16, 16)] = jnp.where(inq, pk - lo, quarter)
                    return cc
                lax.fori_loop(0, RCH // 16, pkb, 0)

                if q == 0:
                    pltpu.sync_copy(ones_v, cnt_sh.at[pkf_v], add=True)
                pltpu.sync_copy(rb_v, acc.at[pkl_v], add=True)
        plsc.subcore_barrier()

        @pl.when(s * ppt < ppq)
        def _():
            gp0 = lo + s * ppn
            lp0 = s * ppn

            pltpu.sync_copy(cnt_sh.at[pl.ds(gp0, ppn)], cred_v)
            pltpu.sync_copy(acc.at[pl.ds(lp0, ppn)], psum_v)

            for pp in range(ppt):
                gpar = (c * 2 + q) * ppq + s * ppt + pp

                def prow(r, carry):
                    vecs = carry[:-1]
                    mcnt = carry[-1]
                    li = pp * n_child + r
                    cnt16 = plsc.load_gather(cred_v,
                                             [jnp.full((16,), li, _i32)])
                    present = cnt16 > 0.0
                    inv16 = jnp.where(present,
                                      1.0 / jnp.maximum(cnt16, 1.0), 0.0)
                    new = []
                    for j in range(D // 16):
                        pj = psum_v[li, pl.ds(j * 16, 16)]
                        tj = tbl_v[gpar, pl.ds(j * 16, 16)]
                        new.append(vecs[j] + jnp.where(present,
                                                       pj * inv16 + tj, 0.0))
                    mcnt = mcnt + jnp.where(present, 1.0, 0.0)
                    return tuple(new) + (mcnt,)

                init = tuple(jnp.zeros((16,), _f32)
                             for _ in range(D // 16 + 1))
                res = lax.fori_loop(0, n_child, prow, init)
                minv = 1.0 / jnp.maximum(res[-1], 1.0)
                for j in range(D // 16):
                    mmrow_v[0, pl.ds(j * 16, 16)] = 1.0 / (
                        1.0 + jnp.exp(-res[j] * minv))
                pltpu.sync_copy(mmrow_v, mm_hbm.at[gpar])
        plsc.subcore_barrier()


def _make_agg(n_parent, n_child):
    nsp = n_parent * n_child
    quarter = nsp // 4
    accr = -(-(quarter + 8) // RCH) * RCH
    ppn = max(1, (n_parent // 4) // NS) * n_child
    return pl.kernel(
        functools.partial(_agg_body, n_parent, n_child, accr),
        out_type=jax.ShapeDtypeStruct((n_parent, 1, D), _f32),
        mesh=_MESH,
        compiler_params=_SCP,
        scratch_types=[
            pltpu.VMEM((nsp // NS,), _f32),
            pltpu.VMEM((ppn,), _f32),
            pltpu.VMEM((RCH,), _i32),
            pltpu.VMEM((RCH,), _i32),
            pltpu.VMEM((RCH,), _i32),
            pltpu.VMEM((RCH,), _i32),
            pltpu.VMEM((RCH,), _f32),
            pltpu.VMEM((RCH, D), _f32),
            pltpu.VMEM((1, D), _f32),
            pltpu.VMEM((n_parent, D), _f32),
            pltpu.VMEM((ppn, D), _f32),
            pltpu.VMEM_SHARED((accr, D), _f32),
            pltpu.VMEM_SHARED((nsp,), _f32),
        ],
    )


_agg_mid = _make_agg(NMID, NSUB)
_agg_big = _make_agg(NBIG, NMID)


def _gat_sc(x, W, asrc2, adst2, src, dst):
    h, hs, hd = _tch(x, W, asrc2, adst2)
    ex, denp = _sca(src, dst, hs, hd)
    inv = _tcinv(denp.reshape(NW, N))
    out4 = _scb(h.reshape(2 * N, DH), src, dst, ex, inv)
    return out4.reshape(N, D)


def kernel(x, W, a_src, a_dst, mid_table, big_table, edge_index, sub_ids,
           mid_ids, big_ids):
    src = edge_index[0]
    dst = edge_index[1]
    asrc2 = a_src.reshape(D, 1)
    adst2 = a_dst.reshape(D, 1)

    sub_emb = _gat_sc(x, W, asrc2, adst2, src, dst)
    mm_mid = _agg_mid(sub_emb, mid_table, sub_ids, mid_ids)
    mid_emb = _gathertc(mm_mid.reshape(NMID, D), mid_ids, NMID)

    mid2 = _gat_sc(mid_emb, W, asrc2, adst2, src, dst)
    mm_big = _agg_big(mid2, big_table, mid_ids, big_ids)
    big_emb = _gathertc(mm_big.reshape(NBIG, D), big_ids, NBIG)

    big2 = _gat_sc(big_emb, W, asrc2, adst2, src, dst)
    return _tcfin(sub_emb, mid2, big2, x)


# speedup vs baseline: 14.7332x; 1.0375x over previous
import functools

import jax
import jax.numpy as jnp
from jax import lax
from jax.experimental import pallas as pl
from jax.experimental.pallas import tpu as pltpu
from jax.experimental.pallas import tpu_sc as plsc

N = 10000
E = 320000
D = 128
DH = D // 2
NSUB = 256
NMID = 64
NBIG = 16

NC = 2
NS = 16
NW = NC * NS
EPT = E // NW
EPS = E // NS
CH = 80
RCH = 80
NRC = N // RCH
KM = (NRC + NS - 1) // NS

_MESH = plsc.VectorSubcoreMesh(core_axis_name="c", subcore_axis_name="s",
                               num_cores=NC, num_subcores=NS)
_SCP = pltpu.CompilerParams(needs_layout_passes=False,
                            use_tc_tiling_on_sc=False)

_f32 = jnp.float32
_i32 = jnp.int32


def _tch_body(x_ref, w_ref, asrc_ref, adst_ref, h_ref, hs_ref, hd_ref):
    h = jnp.dot(x_ref[...], w_ref[...], preferred_element_type=_f32)
    h_ref[...] = h
    hs_ref[...] = jnp.dot(h, asrc_ref[...], preferred_element_type=_f32)
    hd_ref[...] = jnp.dot(h, adst_ref[...], preferred_element_type=_f32)


def _tch(x, W, asrc2, adst2):
    blk = 1000
    h, hs, hd = pl.pallas_call(
        _tch_body,
        grid=(N // blk,),
        in_specs=[pl.BlockSpec((blk, D), lambda i: (i, 0)),
                  pl.BlockSpec((D, D), lambda i: (0, 0)),
                  pl.BlockSpec((D, 1), lambda i: (0, 0)),
                  pl.BlockSpec((D, 1), lambda i: (0, 0))],
        out_specs=[pl.BlockSpec((blk, D), lambda i: (i, 0)),
                   pl.BlockSpec((blk, 1), lambda i: (i, 0)),
                   pl.BlockSpec((blk, 1), lambda i: (i, 0))],
        out_shape=[jax.ShapeDtypeStruct((N, D), _f32),
                   jax.ShapeDtypeStruct((N, 1), _f32),
                   jax.ShapeDtypeStruct((N, 1), _f32)],
    )(x, W, asrc2, adst2)
    return h, hs.reshape(N), hd.reshape(N)


def _tcinv_body(denp_ref, inv_ref):
    inv_ref[...] = 1.0 / (jnp.sum(denp_ref[...], axis=0, keepdims=True) + 1e-16)


def _tcinv(denp):
    inv = pl.pallas_call(
        _tcinv_body,
        out_shape=jax.ShapeDtypeStruct((1, N), _f32),
    )(denp)
    return inv.reshape(N)


def _gathertc_body(n_parent, mm_ref, pid_ref, o_ref):
    ids = pid_ref[0, 0, :]
    onehot = (ids[:, None] ==
              lax.broadcasted_iota(_i32, (ids.shape[0], n_parent), 1)).astype(_f32)
    o_ref[...] = jnp.dot(onehot, mm_ref[...], preferred_element_type=_f32)


def _gathertc(mm, pid, n_parent):
    blk = 1000
    pid3 = pid.reshape(N // blk, 1, blk)
    out = pl.pallas_call(
        functools.partial(_gathertc_body, n_parent),
        grid=(N // blk,),
        in_specs=[pl.BlockSpec((n_parent, D), lambda i: (0, 0)),
                  pl.BlockSpec((1, 1, blk), lambda i: (i, 0, 0))],
        out_specs=pl.BlockSpec((blk, D), lambda i: (i, 0)),
        out_shape=jax.ShapeDtypeStruct((N, D), _f32),
    )(mm, pid3)
    return out


def _tcfin_body(s0, m0, b0, x_ref, o_ref):
    o_ref[0] = s0[...]
    o_ref[1] = m0[...]
    o_ref[2] = b0[...] + x_ref[...]


def _tcfin(sub, mid2, big2, x):
    blk = 1000
    spec = pl.BlockSpec((blk, D), lambda i: (i, 0))
    return pl.pallas_call(
        _tcfin_body,
        grid=(N // blk,),
        in_specs=[spec] * 4,
        out_specs=pl.BlockSpec((3, blk, D), lambda i: (0, i, 0)),
        out_shape=jax.ShapeDtypeStruct((3, N, D), _f32),
    )(sub, mid2, big2, x)


def _sca_body(src_hbm, dst_hbm, hs_hbm, hd_hbm, ex_hbm, denp_hbm,
              hs_v, hd_v, src_v, dst_v, ex_v, den_v):
    c = lax.axis_index("c")
    s = lax.axis_index("s")
    wid = s * NC + c
    base = wid * EPT

    pltpu.sync_copy(hs_hbm, hs_v)
    pltpu.sync_copy(hd_hbm, hd_v)
    pltpu.sync_copy(src_hbm.at[pl.ds(base, EPT)], src_v)
    pltpu.sync_copy(dst_hbm.at[pl.ds(base, EPT)], dst_v)

    z16 = jnp.zeros((16,), _f32)

    def zbody(i, carry):
        den_v[pl.ds(i * 16, 16)] = z16
        return carry
    lax.fori_loop(0, N // 16, zbody, 0)

    def ebody(i, carry):
        s16 = src_v[pl.ds(i * 16, 16)]
        d16 = dst_v[pl.ds(i * 16, 16)]
        a = plsc.load_gather(hs_v, [s16])
        b = plsc.load_gather(hd_v, [d16])
        sc = a + b
        sc = jnp.where(sc >= 0.0, sc, sc * 0.2)
        e16 = jnp.exp(sc)
        ex_v[pl.ds(i * 16, 16)] = e16
        plsc.addupdate_scatter(den_v, [d16], e16)
        return carry
    lax.fori_loop(0, EPT // 16, ebody, 0)

    pltpu.sync_copy(ex_v, ex_hbm.at[pl.ds(base, EPT)])
    pltpu.sync_copy(den_v, denp_hbm.at[wid, 0])


_sca = pl.kernel(
    _sca_body,
    out_type=(jax.ShapeDtypeStruct((E,), _f32),
              jax.ShapeDtypeStruct((NW, 1, N), _f32)),
    mesh=_MESH,
    compiler_params=_SCP,
    scratch_types=[
        pltpu.VMEM((N,), _f32),
        pltpu.VMEM((N,), _f32),
        pltpu.VMEM((EPT,), _i32),
        pltpu.VMEM((EPT,), _i32),
        pltpu.VMEM((EPT,), _f32),
        pltpu.VMEM((N,), _f32),
    ],
)


def _scb_body(h2_hbm, src_hbm, dst_hbm, ex_hbm, inv_hbm, out_hbm,
              inv_v, src_v, dst_v, al_v, dstc_v, rows_v, acc,
              gsem0, gsem1, ssem0, ssem1):
    gsems = (gsem0, gsem1)
    ssems = (ssem0, ssem1)
    c = lax.axis_index("c")
    s = lax.axis_index("s")
    base = s * EPS

    pltpu.sync_copy(inv_hbm, inv_v)
    pltpu.sync_copy(src_hbm.at[pl.ds(base, EPS)], src_v)
    pltpu.sync_copy(dst_hbm.at[pl.ds(base, EPS)], dst_v)
    pltpu.sync_copy(ex_hbm.at[pl.ds(base, EPS)], al_v)

    def abody(i, carry):
        d16 = dst_v[pl.ds(i * 16, 16)]
        iv = plsc.load_gather(inv_v, [d16])
        al_v[pl.ds(i * 16, 16)] = al_v[pl.ds(i * 16, 16)] * iv
        src_v[pl.ds(i * 16, 16)] = src_v[pl.ds(i * 16, 16)] * 2 + c
        return carry
    lax.fori_loop(0, EPS // 16, abody, 0)

    z16 = jnp.zeros((16,), _f32)

    def zrow(i, carry):
        for j in range(DH // 16):
            rows_v[0, i, pl.ds(j * 16, 16)] = z16
        return carry
    lax.fori_loop(0, CH, zrow, 0)
    for k in range(KM):
        ck = s + NS * k

        @pl.when(ck < NRC)
        def _():
            pltpu.sync_copy(rows_v.at[0], acc.at[pl.ds(ck * RCH, RCH)])
    plsc.subcore_barrier()

    nchk = EPS // CH

    def _gather(g, b):
        pltpu.async_copy(h2_hbm.at[src_v.at[pl.ds(g * CH, CH)]],
                         rows_v.at[b], gsems[b])

    def _gather_wait(g, b):
        pltpu.make_async_copy(h2_hbm.at[src_v.at[pl.ds(g * CH, CH)]],
                              rows_v.at[b], gsems[b]).wait()

    def _scatter(b):
        pltpu.async_copy(rows_v.at[b], acc.at[dstc_v.at[b]],
                         ssems[b], add=True)

    def _scatter_wait(b):
        pltpu.make_async_copy(rows_v.at[b], acc.at[dstc_v.at[b]],
                              ssems[b]).wait()

    _gather(0, 0)

    def pair(it, carry):
        for b in range(2):
            g = it * 2 + b
            bp = 1 - b

            @pl.when(g + 1 < nchk)
            def _():
                @pl.when(g >= 1)
                def _():
                    _scatter_wait(bp)
                _gather(g + 1, bp)

            _gather_wait(g, b)

            def cb(k2, cc):
                dstc_v[b, pl.ds(k2 * 16, 16)] = dst_v[pl.ds(g * CH + k2 * 16,
                                                            16)]
                return cc
            lax.fori_loop(0, CH // 16, cb, 0)

            def srow(i2, cc):
                for u in range(2):
                    i = i2 * 2 + u
                    a16 = plsc.load_gather(
                        al_v, [jnp.full((16,), g * CH + i, _i32)])
                    for j in range(DH // 16):
                        rows_v[b, i, pl.ds(j * 16, 16)] = (
                            rows_v[b, i, pl.ds(j * 16, 16)] * a16)
                return cc
            lax.fori_loop(0, CH // 2, srow, 0)

            _scatter(b)
        return carry
    lax.fori_loop(0, nchk // 2, pair, 0)
    _scatter_wait(0)
    _scatter_wait(1)
    plsc.subcore_barrier()

    for k in range(KM):
        ck = s + NS * k

        @pl.when(ck < NRC)
        def _():
            pltpu.sync_copy(acc.at[pl.ds(ck * RCH, RCH)],
                            out_hbm.at[pl.ds(ck * RCH, RCH), c, 0])


_scb = pl.kernel(
    _scb_body,
    out_type=jax.ShapeDtypeStruct((N, NC, 1, DH), _f32),
    mesh=_MESH,
    compiler_params=_SCP,
    scratch_types=[
        pltpu.VMEM((N,), _f32),
        pltpu.VMEM((EPS,), _i32),
        pltpu.VMEM((EPS,), _i32),
        pltpu.VMEM((EPS,), _f32),
        pltpu.VMEM((2, CH), _i32),
        pltpu.VMEM((2, CH, DH), _f32),
        pltpu.VMEM_SHARED((N, DH), _f32),
        pltpu.SemaphoreType.DMA,
        pltpu.SemaphoreType.DMA,
        pltpu.SemaphoreType.DMA,
        pltpu.SemaphoreType.DMA,
    ],
)


def _agg_body(n_parent, n_child, accr,
              p_hbm, tbl_hbm, cid_hbm, pid_hbm, mm_hbm,
              zc_v, cred_v, cidc_v, pidc_v, pkl_v, pkf_v, ones_v,
              rb_v, mmrow_v, tbl_v, psum_v, acc, cnt_sh):
    nsp = n_parent * n_child
    quarter = nsp // 4
    ppq = n_parent // 4
    ppt = max(1, ppq // NS)
    ppn = ppt * n_child
    nzc = (accr + RCH * NS - 1) // (RCH * NS)
    csl = nsp // NS

    c = lax.axis_index("c")
    s = lax.axis_index("s")

    pltpu.sync_copy(tbl_hbm, tbl_v)

    z16 = jnp.zeros((16,), _f32)
    one16 = jnp.ones((16,), _f32)

    def zrow(i, carry):
        for j in range(D // 16):
            rb_v[i, pl.ds(j * 16, 16)] = z16
        return carry
    lax.fori_loop(0, RCH, zrow, 0)

    def zo(i, carry):
        ones_v[pl.ds(i * 16, 16)] = one16
        return carry
    lax.fori_loop(0, RCH // 16, zo, 0)

    for q in range(2):
        for k in range(nzc):
            ck = s + NS * k

            @pl.when(ck * RCH < accr)
            def _():
                pltpu.sync_copy(rb_v, acc.at[pl.ds(ck * RCH, RCH)])
        if q == 0:
            def zc(i, carry):
                zc_v[pl.ds(i * 16, 16)] = z16
                return carry
            lax.fori_loop(0, csl // 16, zc, 0)
            pltpu.sync_copy(zc_v, cnt_sh.at[pl.ds(s * csl, csl)])
        plsc.subcore_barrier()

        lo = (c * 2 + q) * quarter
        for k in range(KM):
            ck = s + NS * k

            @pl.when(ck < NRC)
            def _():
                r0 = ck * RCH
                pltpu.sync_copy(cid_hbm.at[pl.ds(r0, RCH)], cidc_v)
                pltpu.sync_copy(pid_hbm.at[pl.ds(r0, RCH)], pidc_v)
                pltpu.sync_copy(p_hbm.at[pl.ds(r0, RCH)], rb_v)

                def pkb(t, cc):
                    ci = cidc_v[pl.ds(t * 16, 16)]
                    pi = pidc_v[pl.ds(t * 16, 16)]
                    pk = pi * n_child + ci
                    pkf_v[pl.ds(t * 16, 16)] = pk
                    inq = (pk >= lo) & (pk < lo + quarter)
                    pkl_v[pl.ds(t * 16, 16)] = jnp.where(inq, pk - lo, quarter)
                    return cc
                lax.fori_loop(0, RCH // 16, pkb, 0)

                if q == 0:
                    pltpu.sync_copy(ones_v, cnt_sh.at[pkf_v], add=True)
                pltpu.sync_copy(rb_v, acc.at[pkl_v], add=True)
        plsc.subcore_barrier()

        @pl.when(s * ppt < ppq)
        def _():
            gp0 = lo + s * ppn
            lp0 = s * ppn

            pltpu.sync_copy(cnt_sh.at[pl.ds(gp0, ppn)], cred_v)
            pltpu.sync_copy(acc.at[pl.ds(lp0, ppn)], psum_v)

            for pp in range(ppt):
                gpar = (c * 2 + q) * ppq + s * ppt + pp

                def prow(r, carry):
                    vecs = carry[:-1]
                    mcnt = carry[-1]
                    li = pp * n_child + r
                    cnt16 = plsc.load_gather(cred_v,
                                             [jnp.full((16,), li, _i32)])
                    present = cnt16 > 0.0
                    inv16 = jnp.where(present,
                                      1.0 / jnp.maximum(cnt16, 1.0), 0.0)
                    new = []
                    for j in range(D // 16):
                        pj = psum_v[li, pl.ds(j * 16, 16)]
                        tj = tbl_v[gpar, pl.ds(j * 16, 16)]
                        new.append(vecs[j] + jnp.where(present,
                                                       pj * inv16 + tj, 0.0))
                    mcnt = mcnt + jnp.where(present, 1.0, 0.0)
                    return tuple(new) + (mcnt,)

                init = tuple(jnp.zeros((16,), _f32)
                             for _ in range(D // 16 + 1))
                res = lax.fori_loop(0, n_child, prow, init)
                minv = 1.0 / jnp.maximum(res[-1], 1.0)
                for j in range(D // 16):
                    mmrow_v[0, pl.ds(j * 16, 16)] = 1.0 / (
                        1.0 + jnp.exp(-res[j] * minv))
                pltpu.sync_copy(mmrow_v, mm_hbm.at[gpar])
        plsc.subcore_barrier()


def _make_agg(n_parent, n_child):
    nsp = n_parent * n_child
    quarter = nsp // 4
    accr = -(-(quarter + 8) // RCH) * RCH
    ppn = max(1, (n_parent // 4) // NS) * n_child
    return pl.kernel(
        functools.partial(_agg_body, n_parent, n_child, accr),
        out_type=jax.ShapeDtypeStruct((n_parent, 1, D), _f32),
        mesh=_MESH,
        compiler_params=_SCP,
        scratch_types=[
            pltpu.VMEM((nsp // NS,), _f32),
            pltpu.VMEM((ppn,), _f32),
            pltpu.VMEM((RCH,), _i32),
            pltpu.VMEM((RCH,), _i32),
            pltpu.VMEM((RCH,), _i32),
            pltpu.VMEM((RCH,), _i32),
            pltpu.VMEM((RCH,), _f32),
            pltpu.VMEM((RCH, D), _f32),
            pltpu.VMEM((1, D), _f32),
            pltpu.VMEM((n_parent, D), _f32),
            pltpu.VMEM((ppn, D), _f32),
            pltpu.VMEM_SHARED((accr, D), _f32),
            pltpu.VMEM_SHARED((nsp,), _f32),
        ],
    )


_agg_mid = _make_agg(NMID, NSUB)
_agg_big = _make_agg(NBIG, NMID)


def _gat_sc(x, W, asrc2, adst2, src, dst):
    h, hs, hd = _tch(x, W, asrc2, adst2)
    ex, denp = _sca(src, dst, hs, hd)
    inv = _tcinv(denp.reshape(NW, N))
    out4 = _scb(h.reshape(2 * N, DH), src, dst, ex, inv)
    return out4.reshape(N, D)


def kernel(x, W, a_src, a_dst, mid_table, big_table, edge_index, sub_ids,
           mid_ids, big_ids):
    src = edge_index[0]
    dst = edge_index[1]
    asrc2 = a_src.reshape(D, 1)
    adst2 = a_dst.reshape(D, 1)

    sub_emb = _gat_sc(x, W, asrc2, adst2, src, dst)
    mm_mid = _agg_mid(sub_emb, mid_table, sub_ids, mid_ids)
    mid_emb = _gathertc(mm_mid.reshape(NMID, D), mid_ids, NMID)

    mid2 = _gat_sc(mid_emb, W, asrc2, adst2, src, dst)
    mm_big = _agg_big(mid2, big_table, mid_ids, big_ids)
    big_emb = _gathertc(mm_big.reshape(NBIG, D), big_ids, NBIG)

    big2 = _gat_sc(big_emb, W, asrc2, adst2, src, dst)
    return _tcfin(sub_emb, mid2, big2, x)


# speedup vs baseline: 14.7753x; 1.0029x over previous
import functools

import jax
import jax.numpy as jnp
from jax import lax
from jax.experimental import pallas as pl
from jax.experimental.pallas import tpu as pltpu
from jax.experimental.pallas import tpu_sc as plsc

N = 10000
E = 320000
D = 128
DH = D // 2
NSUB = 256
NMID = 64
NBIG = 16

NC = 2
NS = 16
NW = NC * NS
EPT = E // NW
EPS = E // NS
CH = 80
RCH = 80
NRC = N // RCH
KM = (NRC + NS - 1) // NS

_MESH = plsc.VectorSubcoreMesh(core_axis_name="c", subcore_axis_name="s",
                               num_cores=NC, num_subcores=NS)
_SCP = pltpu.CompilerParams(needs_layout_passes=False,
                            use_tc_tiling_on_sc=False)

_f32 = jnp.float32
_i32 = jnp.int32


def _tch_body(x_ref, w_ref, asrc_ref, adst_ref, h_ref, hs_ref, hd_ref):
    h = jnp.dot(x_ref[...], w_ref[...], preferred_element_type=_f32)
    h_ref[...] = h
    hs_ref[...] = jnp.dot(h, asrc_ref[...], preferred_element_type=_f32)
    hd_ref[...] = jnp.dot(h, adst_ref[...], preferred_element_type=_f32)


def _tch(x, W, asrc2, adst2):
    blk = 1000
    h, hs, hd = pl.pallas_call(
        _tch_body,
        grid=(N // blk,),
        in_specs=[pl.BlockSpec((blk, D), lambda i: (i, 0)),
                  pl.BlockSpec((D, D), lambda i: (0, 0)),
                  pl.BlockSpec((D, 1), lambda i: (0, 0)),
                  pl.BlockSpec((D, 1), lambda i: (0, 0))],
        out_specs=[pl.BlockSpec((blk, D), lambda i: (i, 0)),
                   pl.BlockSpec((blk, 1), lambda i: (i, 0)),
                   pl.BlockSpec((blk, 1), lambda i: (i, 0))],
        out_shape=[jax.ShapeDtypeStruct((N, D), _f32),
                   jax.ShapeDtypeStruct((N, 1), _f32),
                   jax.ShapeDtypeStruct((N, 1), _f32)],
    )(x, W, asrc2, adst2)
    return h, hs.reshape(N), hd.reshape(N)


def _tcinv_body(denp_ref, inv_ref):
    inv_ref[...] = 1.0 / (jnp.sum(denp_ref[...], axis=0, keepdims=True) + 1e-16)


def _tcinv(denp):
    inv = pl.pallas_call(
        _tcinv_body,
        out_shape=jax.ShapeDtypeStruct((1, N), _f32),
    )(denp)
    return inv.reshape(N)


def _gathertc_body(n_parent, mm_ref, pid_ref, o_ref):
    ids = pid_ref[0, 0, :]
    onehot = (ids[:, None] ==
              lax.broadcasted_iota(_i32, (ids.shape[0], n_parent), 1)).astype(_f32)
    o_ref[...] = jnp.dot(onehot, mm_ref[...], preferred_element_type=_f32)


def _gathertc(mm, pid, n_parent):
    blk = 1000
    pid3 = pid.reshape(N // blk, 1, blk)
    out = pl.pallas_call(
        functools.partial(_gathertc_body, n_parent),
        grid=(N // blk,),
        in_specs=[pl.BlockSpec((n_parent, D), lambda i: (0, 0)),
                  pl.BlockSpec((1, 1, blk), lambda i: (i, 0, 0))],
        out_specs=pl.BlockSpec((blk, D), lambda i: (i, 0)),
        out_shape=jax.ShapeDtypeStruct((N, D), _f32),
    )(mm, pid3)
    return out


def _tcfin_body(s0, m0, b0, x_ref, o_ref):
    o_ref[0] = s0[...]
    o_ref[1] = m0[...]
    o_ref[2] = b0[...] + x_ref[...]


def _tcfin(sub, mid2, big2, x):
    blk = 1000
    spec = pl.BlockSpec((blk, D), lambda i: (i, 0))
    return pl.pallas_call(
        _tcfin_body,
        grid=(N // blk,),
        in_specs=[spec] * 4,
        out_specs=pl.BlockSpec((3, blk, D), lambda i: (0, i, 0)),
        out_shape=jax.ShapeDtypeStruct((3, N, D), _f32),
    )(sub, mid2, big2, x)


def _sca_body(src_hbm, dst_hbm, hs_hbm, hd_hbm, ex_hbm, denp_hbm,
              hs_v, hd_v, src_v, dst_v, ex_v, den_v):
    c = lax.axis_index("c")
    s = lax.axis_index("s")
    wid = s * NC + c
    base = wid * EPT

    pltpu.sync_copy(hs_hbm, hs_v)
    pltpu.sync_copy(hd_hbm, hd_v)
    pltpu.sync_copy(src_hbm.at[pl.ds(base, EPT)], src_v)
    pltpu.sync_copy(dst_hbm.at[pl.ds(base, EPT)], dst_v)

    z16 = jnp.zeros((16,), _f32)

    def zbody(i, carry):
        den_v[pl.ds(i * 16, 16)] = z16
        return carry
    lax.fori_loop(0, N // 16, zbody, 0)

    def ebody(i, carry):
        s16 = src_v[pl.ds(i * 16, 16)]
        d16 = dst_v[pl.ds(i * 16, 16)]
        a = plsc.load_gather(hs_v, [s16])
        b = plsc.load_gather(hd_v, [d16])
        sc = a + b
        sc = jnp.where(sc >= 0.0, sc, sc * 0.2)
        e16 = jnp.exp(sc)
        ex_v[pl.ds(i * 16, 16)] = e16
        plsc.addupdate_scatter(den_v, [d16], e16)
        return carry
    lax.fori_loop(0, EPT // 16, ebody, 0)

    pltpu.sync_copy(ex_v, ex_hbm.at[pl.ds(base, EPT)])
    pltpu.sync_copy(den_v, denp_hbm.at[wid, 0])


_sca = pl.kernel(
    _sca_body,
    out_type=(jax.ShapeDtypeStruct((E,), _f32),
              jax.ShapeDtypeStruct((NW, 1, N), _f32)),
    mesh=_MESH,
    compiler_params=_SCP,
    scratch_types=[
        pltpu.VMEM((N,), _f32),
        pltpu.VMEM((N,), _f32),
        pltpu.VMEM((EPT,), _i32),
        pltpu.VMEM((EPT,), _i32),
        pltpu.VMEM((EPT,), _f32),
        pltpu.VMEM((N,), _f32),
    ],
)


def _scb_body(h2_hbm, src_hbm, dst_hbm, ex_hbm, inv_hbm, out_hbm,
              inv_v, src_v, dst_v, al_v, dstc_v, rows_v, acc,
              gsem0, gsem1, ssem0, ssem1):
    gsems = (gsem0, gsem1)
    ssems = (ssem0, ssem1)
    c = lax.axis_index("c")
    s = lax.axis_index("s")
    base = s * EPS

    pltpu.sync_copy(inv_hbm, inv_v)
    pltpu.sync_copy(src_hbm.at[pl.ds(base, EPS)], src_v)
    pltpu.sync_copy(dst_hbm.at[pl.ds(base, EPS)], dst_v)
    pltpu.sync_copy(ex_hbm.at[pl.ds(base, EPS)], al_v)

    def abody(i, carry):
        d16 = dst_v[pl.ds(i * 16, 16)]
        iv = plsc.load_gather(inv_v, [d16])
        al_v[pl.ds(i * 16, 16)] = al_v[pl.ds(i * 16, 16)] * iv
        src_v[pl.ds(i * 16, 16)] = src_v[pl.ds(i * 16, 16)] * 2 + c
        return carry
    lax.fori_loop(0, EPS // 16, abody, 0)

    z16 = jnp.zeros((16,), _f32)

    def zrow(i, carry):
        for j in range(DH // 16):
            rows_v[0, i, pl.ds(j * 16, 16)] = z16
        return carry
    lax.fori_loop(0, CH, zrow, 0)
    for k in range(KM):
        ck = s + NS * k

        @pl.when(ck < NRC)
        def _():
            pltpu.sync_copy(rows_v.at[0], acc.at[pl.ds(ck * RCH, RCH)])
    plsc.subcore_barrier()

    nchk = EPS // CH

    def _gather(g, b):
        pltpu.async_copy(h2_hbm.at[src_v.at[pl.ds(g * CH, CH)]],
                         rows_v.at[b], gsems[b])

    def _gather_wait(g, b):
        pltpu.make_async_copy(h2_hbm.at[src_v.at[pl.ds(g * CH, CH)]],
                              rows_v.at[b], gsems[b]).wait()

    def _scatter(b):
        pltpu.async_copy(rows_v.at[b], acc.at[dstc_v.at[b]],
                         ssems[b], add=True)

    def _scatter_wait(b):
        pltpu.make_async_copy(rows_v.at[b], acc.at[dstc_v.at[b]],
                              ssems[b]).wait()

    _gather(0, 0)

    def pair(it, carry):
        for b in range(2):
            g = it * 2 + b
            bp = 1 - b

            @pl.when(g + 1 < nchk)
            def _():
                @pl.when(g >= 1)
                def _():
                    _scatter_wait(bp)
                _gather(g + 1, bp)

            _gather_wait(g, b)

            def cb(k2, cc):
                dstc_v[b, pl.ds(k2 * 16, 16)] = dst_v[pl.ds(g * CH + k2 * 16,
                                                            16)]
                return cc
            lax.fori_loop(0, CH // 16, cb, 0)

            def srow(i2, cc):
                for u in range(8):
                    i = i2 * 8 + u
                    a16 = plsc.load_gather(
                        al_v, [jnp.full((16,), g * CH + i, _i32)])
                    for j in range(DH // 16):
                        rows_v[b, i, pl.ds(j * 16, 16)] = (
                            rows_v[b, i, pl.ds(j * 16, 16)] * a16)
                return cc
            lax.fori_loop(0, CH // 8, srow, 0)

            _scatter(b)
        return carry
    lax.fori_loop(0, nchk // 2, pair, 0)
    _scatter_wait(0)
    _scatter_wait(1)
    plsc.subcore_barrier()

    for k in range(KM):
        ck = s + NS * k

        @pl.when(ck < NRC)
        def _():
            pltpu.sync_copy(acc.at[pl.ds(ck * RCH, RCH)],
                            out_hbm.at[pl.ds(ck * RCH, RCH), c, 0])


_scb = pl.kernel(
    _scb_body,
    out_type=jax.ShapeDtypeStruct((N, NC, 1, DH), _f32),
    mesh=_MESH,
    compiler_params=_SCP,
    scratch_types=[
        pltpu.VMEM((N,), _f32),
        pltpu.VMEM((EPS,), _i32),
        pltpu.VMEM((EPS,), _i32),
        pltpu.VMEM((EPS,), _f32),
        pltpu.VMEM((2, CH), _i32),
        pltpu.VMEM((2, CH, DH), _f32),
        pltpu.VMEM_SHARED((N, DH), _f32),
        pltpu.SemaphoreType.DMA,
        pltpu.SemaphoreType.DMA,
        pltpu.SemaphoreType.DMA,
        pltpu.SemaphoreType.DMA,
    ],
)


def _agg_body(n_parent, n_child, accr,
              p_hbm, tbl_hbm, cid_hbm, pid_hbm, mm_hbm,
              zc_v, cred_v, cidc_v, pidc_v, pkl_v, pkf_v, ones_v,
              rb_v, mmrow_v, tbl_v, psum_v, acc, cnt_sh):
    nsp = n_parent * n_child
    quarter = nsp // 4
    ppq = n_parent // 4
    ppt = max(1, ppq // NS)
    ppn = ppt * n_child
    nzc = (accr + RCH * NS - 1) // (RCH * NS)
    csl = nsp // NS

    c = lax.axis_index("c")
    s = lax.axis_index("s")

    pltpu.sync_copy(tbl_hbm, tbl_v)

    z16 = jnp.zeros((16,), _f32)
    one16 = jnp.ones((16,), _f32)

    def zrow(i, carry):
        for j in range(D // 16):
            rb_v[i, pl.ds(j * 16, 16)] = z16
        return carry
    lax.fori_loop(0, RCH, zrow, 0)

    def zo(i, carry):
        ones_v[pl.ds(i * 16, 16)] = one16
        return carry
    lax.fori_loop(0, RCH // 16, zo, 0)

    for q in range(2):
        for k in range(nzc):
            ck = s + NS * k

            @pl.when(ck * RCH < accr)
            def _():
                pltpu.sync_copy(rb_v, acc.at[pl.ds(ck * RCH, RCH)])
        if q == 0:
            def zc(i, carry):
                zc_v[pl.ds(i * 16, 16)] = z16
                return carry
            lax.fori_loop(0, csl // 16, zc, 0)
            pltpu.sync_copy(zc_v, cnt_sh.at[pl.ds(s * csl, csl)])
        plsc.subcore_barrier()

        lo = (c * 2 + q) * quarter
        for k in range(KM):
            ck = s + NS * k

            @pl.when(ck < NRC)
            def _():
                r0 = ck * RCH
                pltpu.sync_copy(cid_hbm.at[pl.ds(r0, RCH)], cidc_v)
                pltpu.sync_copy(pid_hbm.at[pl.ds(r0, RCH)], pidc_v)
                pltpu.sync_copy(p_hbm.at[pl.ds(r0, RCH)], rb_v)

                def pkb(t, cc):
                    ci = cidc_v[pl.ds(t * 16, 16)]
                    pi = pidc_v[pl.ds(t * 16, 16)]
                    pk = pi * n_child + ci
                    pkf_v[pl.ds(t * 16, 16)] = pk
                    inq = (pk >= lo) & (pk < lo + quarter)
                    pkl_v[pl.ds(t * 16, 16)] = jnp.where(inq, pk - lo, quarter)
                    return cc
                lax.fori_loop(0, RCH // 16, pkb, 0)

                if q == 0:
                    pltpu.sync_copy(ones_v, cnt_sh.at[pkf_v], add=True)
                pltpu.sync_copy(rb_v, acc.at[pkl_v], add=True)
        plsc.subcore_barrier()

        @pl.when(s * ppt < ppq)
        def _():
            gp0 = lo + s * ppn
            lp0 = s * ppn

            pltpu.sync_copy(cnt_sh.at[pl.ds(gp0, ppn)], cred_v)
            pltpu.sync_copy(acc.at[pl.ds(lp0, ppn)], psum_v)

            for pp in range(ppt):
                gpar = (c * 2 + q) * ppq + s * ppt + pp

                def prow(r, carry):
                    vecs = carry[:-1]
                    mcnt = carry[-1]
                    li = pp * n_child + r
                    cnt16 = plsc.load_gather(cred_v,
                                             [jnp.full((16,), li, _i32)])
                    present = cnt16 > 0.0
                    inv16 = jnp.where(present,
                                      1.0 / jnp.maximum(cnt16, 1.0), 0.0)
                    new = []
                    for j in range(D // 16):
                        pj = psum_v[li, pl.ds(j * 16, 16)]
                        tj = tbl_v[gpar, pl.ds(j * 16, 16)]
                        new.append(vecs[j] + jnp.where(present,
                                                       pj * inv16 + tj, 0.0))
                    mcnt = mcnt + jnp.where(present, 1.0, 0.0)
                    return tuple(new) + (mcnt,)

                init = tuple(jnp.zeros((16,), _f32)
                             for _ in range(D // 16 + 1))
                res = lax.fori_loop(0, n_child, prow, init)
                minv = 1.0 / jnp.maximum(res[-1], 1.0)
                for j in range(D // 16):
                    mmrow_v[0, pl.ds(j * 16, 16)] = 1.0 / (
                        1.0 + jnp.exp(-res[j] * minv))
                pltpu.sync_copy(mmrow_v, mm_hbm.at[gpar])
        plsc.subcore_barrier()


def _make_agg(n_parent, n_child):
    nsp = n_parent * n_child
    quarter = nsp // 4
    accr = -(-(quarter + 8) // RCH) * RCH
    ppn = max(1, (n_parent // 4) // NS) * n_child
    return pl.kernel(
        functools.partial(_agg_body, n_parent, n_child, accr),
        out_type=jax.ShapeDtypeStruct((n_parent, 1, D), _f32),
        mesh=_MESH,
        compiler_params=_SCP,
        scratch_types=[
            pltpu.VMEM((nsp // NS,), _f32),
            pltpu.VMEM((ppn,), _f32),
            pltpu.VMEM((RCH,), _i32),
            pltpu.VMEM((RCH,), _i32),
            pltpu.VMEM((RCH,), _i32),
            pltpu.VMEM((RCH,), _i32),
            pltpu.VMEM((RCH,), _f32),
            pltpu.VMEM((RCH, D), _f32),
            pltpu.VMEM((1, D), _f32),
            pltpu.VMEM((n_parent, D), _f32),
            pltpu.VMEM((ppn, D), _f32),
            pltpu.VMEM_SHARED((accr, D), _f32),
            pltpu.VMEM_SHARED((nsp,), _f32),
        ],
    )


_agg_mid = _make_agg(NMID, NSUB)
_agg_big = _make_agg(NBIG, NMID)


def _gat_sc(x, W, asrc2, adst2, src, dst):
    h, hs, hd = _tch(x, W, asrc2, adst2)
    ex, denp = _sca(src, dst, hs, hd)
    inv = _tcinv(denp.reshape(NW, N))
    out4 = _scb(h.reshape(2 * N, DH), src, dst, ex, inv)
    return out4.reshape(N, D)


def kernel(x, W, a_src, a_dst, mid_table, big_table, edge_index, sub_ids,
           mid_ids, big_ids):
    src = edge_index[0]
    dst = edge_index[1]
    asrc2 = a_src.reshape(D, 1)
    adst2 = a_dst.reshape(D, 1)

    sub_emb = _gat_sc(x, W, asrc2, adst2, src, dst)
    mm_mid = _agg_mid(sub_emb, mid_table, sub_ids, mid_ids)
    mid_emb = _gathertc(mm_mid.reshape(NMID, D), mid_ids, NMID)

    mid2 = _gat_sc(mid_emb, W, asrc2, adst2, src, dst)
    mm_big = _agg_big(mid2, big_table, mid_ids, big_ids)
    big_emb = _gathertc(mm_big.reshape(NBIG, D), big_ids, NBIG)

    big2 = _gat_sc(big_emb, W, asrc2, adst2, src, dst)
    return _tcfin(sub_emb, mid2, big2, x)
